# baseline (speedup 1.0000x reference)
import jax
import jax.numpy as jnp
from jax import lax
from jax.experimental import pallas as pl

_G = 8
_CH = 512


def _body(feat_ref, hid_ref, w4_ref, wa4_ref, bf_ref, wh_ref, bh_ref,
          ba_ref, wb_ref, bb_ref, ctx_ref, alpha_ref, lp_ref):
    G, L, D = feat_ref.shape
    A = bf_ref.shape[1]
    npk = w4_ref.shape[0]
    rows_grp = npk * _CH
    g_grp = rows_grp // L
    X = feat_ref[...].reshape(G * L, D)
    hh = hid_ref[...]
    BQ = jnp.dot(hh, wh_ref[...]) + bh_ref[...] + bf_ref[...]
    Beta = jax.nn.sigmoid(jnp.dot(hh, wb_ref[...]) + bb_ref[...])
    iota = (lax.broadcasted_iota(jnp.int32, (L // _CH, _CH), 0) * _CH
            + lax.broadcasted_iota(jnp.int32, (L // _CH, _CH), 1))
    for j in range(G * L // rows_grp):
        g0 = j * g_grp
        base = j * rows_grp
        U = jnp.dot(X[base:base + _CH], w4_ref[0])
        for k in range(1, npk):
            U = U + jnp.dot(X[base + k * _CH:base + (k + 1) * _CH],
                            w4_ref[k])
        br = jnp.concatenate(
            [BQ[g0 + (k * _CH) // L:g0 + (k * _CH) // L + 1]
             for k in range(npk)], axis=1)
        T = jnp.tanh(U + br)
        E4 = lax.dot_general(wa4_ref[...], T,
                             (((0,), (1,)), ((), ()))) + ba_ref[...]
        for gl in range(g_grp):
            g = g0 + gl
            e = E4[gl * (L // _CH):(gl + 1) * (L // _CH)]
            m = jnp.max(e)
            p = jnp.exp(e - m)
            s = jnp.sum(p)
            alpha = p / s
            amax = jnp.max(alpha)
            idx = jnp.min(jnp.where(alpha == amax, iota, L))
            row = feat_ref[g, pl.ds(idx, 1), :]
            ctx_ref[pl.ds(g, 1), :] = row * Beta[g:g + 1]
            alpha_ref[g] = alpha
            lp_ref[pl.ds(g, 1), :] = jnp.log(amax).reshape(1, 1)


def kernel(features, hidden, Wf, bf, Wh, bh, Wa, ba, Wb, bb):
    B, L, D = features.shape
    H = hidden.shape[1]
    A = Wf.shape[1]
    f32 = jnp.float32
    G = _G
    NP = 128 // A
    W4 = jnp.stack([jnp.pad(Wf, ((0, 0), (A * k, 128 - A * (k + 1))))
                    for k in range(NP)])
    Wa4 = jnp.concatenate(
        [jnp.pad(Wa, ((A * k, 128 - A * (k + 1)), (0, 0)))
         for k in range(NP)], axis=1)
    ctx, alpha, lp = pl.pallas_call(
        _body,
        grid=(B // G,),
        in_specs=[
            pl.BlockSpec((G, L, D), lambda b: (b, 0, 0)),
            pl.BlockSpec((G, H), lambda b: (b, 0)),
            pl.BlockSpec((NP, D, 128), lambda b: (0, 0, 0)),
            pl.BlockSpec((128, NP), lambda b: (0, 0)),
            pl.BlockSpec((1, A), lambda b: (0, 0)),
            pl.BlockSpec((H, A), lambda b: (0, 0)),
            pl.BlockSpec((1, A), lambda b: (0, 0)),
            pl.BlockSpec((1, 1), lambda b: (0, 0)),
            pl.BlockSpec((H, 1), lambda b: (0, 0)),
            pl.BlockSpec((1, 1), lambda b: (0, 0)),
        ],
        out_specs=[
            pl.BlockSpec((G, D), lambda b: (b, 0)),
            pl.BlockSpec((G, L // _CH, _CH), lambda b: (b, 0, 0)),
            pl.BlockSpec((G, 1), lambda b: (b, 0)),
        ],
        out_shape=[
            jax.ShapeDtypeStruct((B, D), f32),
            jax.ShapeDtypeStruct((B, L // _CH, _CH), f32),
            jax.ShapeDtypeStruct((B, 1), f32),
        ],
    )(features, hidden, W4, Wa4, bf.reshape(1, A), Wh,
      bh.reshape(1, A), ba.reshape(1, 1), Wb, bb.reshape(1, 1))
    return ctx, alpha.reshape(B, L), lp.reshape(B)


# speedup vs baseline: 1.0930x; 1.0930x over previous
import jax
import jax.numpy as jnp
from jax import lax
from jax.experimental import pallas as pl

_G = 8
_CH = 512


def _body(feat_ref, hid_ref, w4_ref, wa4_ref, bf_ref, wh_ref, bh_ref,
          ba_ref, wb_ref, bb_ref, ctx_ref, alpha_ref, lp_ref):
    G, L, D = feat_ref.shape
    A = bf_ref.shape[1]
    npk = w4_ref.shape[0]
    rows_grp = npk * _CH
    g_grp = rows_grp // L
    ng = G * L // rows_grp
    X = feat_ref[...].reshape(G * L, D)
    hh = hid_ref[...]
    BQ = jnp.dot(hh, wh_ref[...]) + bh_ref[...] + bf_ref[...]
    Beta = jax.nn.sigmoid(jnp.dot(hh, wb_ref[...]) + bb_ref[...])
    parts = []
    for j in range(ng):
        g0 = j * g_grp
        base = j * rows_grp
        U = jnp.dot(X[base:base + _CH], w4_ref[0])
        for k in range(1, npk):
            U = U + jnp.dot(X[base + k * _CH:base + (k + 1) * _CH],
                            w4_ref[k])
        br = jnp.concatenate(
            [BQ[g0 + (k * _CH) // L:g0 + (k * _CH) // L + 1]
             for k in range(npk)], axis=1)
        T = jnp.tanh(U + br)
        parts.append(jnp.transpose(jnp.dot(T, wa4_ref[...])))
    nr = L // _CH
    E3 = (jnp.concatenate(parts, axis=0).reshape(G, nr, _CH)
          + ba_ref[...])
    m = jnp.max(jnp.max(E3, axis=2, keepdims=True), axis=1, keepdims=True)
    p = jnp.exp(E3 - m)
    s = jnp.sum(jnp.sum(p, axis=2, keepdims=True), axis=1, keepdims=True)
    alpha3 = p / s
    amax = jnp.max(jnp.max(alpha3, axis=2, keepdims=True), axis=1,
                   keepdims=True)
    iota3 = (lax.broadcasted_iota(jnp.int32, E3.shape, 1) * _CH
             + lax.broadcasted_iota(jnp.int32, E3.shape, 2))
    idxv = jnp.min(jnp.min(jnp.where(alpha3 == amax, iota3, L), axis=2),
                   axis=1)
    alpha_ref[...] = alpha3
    lp_ref[...] = jnp.log(amax).reshape(G, 1)
    for g in range(G):
        row = feat_ref[g, pl.ds(idxv[g], 1), :]
        ctx_ref[pl.ds(g, 1), :] = row * Beta[g:g + 1]


def kernel(features, hidden, Wf, bf, Wh, bh, Wa, ba, Wb, bb):
    B, L, D = features.shape
    H = hidden.shape[1]
    A = Wf.shape[1]
    f32 = jnp.float32
    G = _G
    NP = 128 // A
    W4 = jnp.stack([jnp.pad(Wf, ((0, 0), (A * k, 128 - A * (k + 1))))
                    for k in range(NP)])
    Wa4 = sum(jnp.pad(Wa, ((A * k, 128 - A * (k + 1)),
                           (k, NP - k - 1))) for k in range(NP))
    ctx, alpha, lp = pl.pallas_call(
        _body,
        grid=(B // G,),
        in_specs=[
            pl.BlockSpec((G, L, D), lambda b: (b, 0, 0)),
            pl.BlockSpec((G, H), lambda b: (b, 0)),
            pl.BlockSpec((NP, D, 128), lambda b: (0, 0, 0)),
            pl.BlockSpec((128, NP), lambda b: (0, 0)),
            pl.BlockSpec((1, A), lambda b: (0, 0)),
            pl.BlockSpec((H, A), lambda b: (0, 0)),
            pl.BlockSpec((1, A), lambda b: (0, 0)),
            pl.BlockSpec((1, 1), lambda b: (0, 0)),
            pl.BlockSpec((H, 1), lambda b: (0, 0)),
            pl.BlockSpec((1, 1), lambda b: (0, 0)),
        ],
        out_specs=[
            pl.BlockSpec((G, D), lambda b: (b, 0)),
            pl.BlockSpec((G, L // _CH, _CH), lambda b: (b, 0, 0)),
            pl.BlockSpec((G, 1), lambda b: (b, 0)),
        ],
        out_shape=[
            jax.ShapeDtypeStruct((B, D), f32),
            jax.ShapeDtypeStruct((B, L // _CH, _CH), f32),
            jax.ShapeDtypeStruct((B, 1), f32),
        ],
    )(features, hidden, W4, Wa4, bf.reshape(1, A), Wh,
      bh.reshape(1, A), ba.reshape(1, 1), Wb, bb.reshape(1, 1))
    return ctx, alpha.reshape(B, L), lp.reshape(B)


# speedup vs baseline: 1.1299x; 1.0338x over previous
import jax
import jax.numpy as jnp
from jax import lax
from jax.experimental import pallas as pl

_G = 8
_CH = 512


def _body(feat_ref, hid_ref, w4_ref, wa4_ref, bf_ref, wh_ref, bh_ref,
          ba_ref, wb_ref, bb_ref, ctx_ref, alpha_ref, lp_ref):
    G, L, D = feat_ref.shape
    A = bf_ref.shape[1]
    npk = w4_ref.shape[0]
    rows_grp = npk * _CH
    g_grp = rows_grp // L
    ng = G * L // rows_grp
    X = feat_ref[...].reshape(G * L, D)
    hh = hid_ref[...]
    BQ = jnp.dot(hh, wh_ref[...]) + bh_ref[...] + bf_ref[...]
    Beta = jax.nn.sigmoid(jnp.dot(hh, wb_ref[...]) + bb_ref[...])
    parts = []
    for j in range(ng):
        g0 = j * g_grp
        base = j * rows_grp
        U = jnp.dot(X[base:base + _CH], w4_ref[0])
        for k in range(1, npk):
            U = U + jnp.dot(X[base + k * _CH:base + (k + 1) * _CH],
                            w4_ref[k])
        br = jnp.concatenate(
            [BQ[g0 + (k * _CH) // L:g0 + (k * _CH) // L + 1]
             for k in range(npk)], axis=1)
        T = jnp.tanh(U + br)
        parts.append(jnp.transpose(jnp.dot(T, wa4_ref[...])))
    E2 = (jnp.concatenate(parts, axis=0).reshape(G, L)
          + ba_ref[...])
    m = jnp.max(E2, axis=1, keepdims=True)
    p = jnp.exp(E2 - m)
    s = jnp.sum(p, axis=1, keepdims=True)
    alpha2 = p / s
    amax = jnp.max(alpha2, axis=1, keepdims=True)
    iota2 = lax.broadcasted_iota(jnp.int32, E2.shape, 1)
    idxv = jnp.min(jnp.where(alpha2 == amax, iota2, L),
                   axis=1)
    alpha_ref[...] = alpha2
    lp_ref[...] = jnp.log(amax)
    for g in range(G):
        row = feat_ref[g, pl.ds(idxv[g], 1), :]
        ctx_ref[pl.ds(g, 1), :] = row * Beta[g:g + 1]


def kernel(features, hidden, Wf, bf, Wh, bh, Wa, ba, Wb, bb):
    B, L, D = features.shape
    H = hidden.shape[1]
    A = Wf.shape[1]
    f32 = jnp.float32
    G = _G
    NP = 128 // A
    W4 = jnp.stack([jnp.pad(Wf, ((0, 0), (A * k, 128 - A * (k + 1))))
                    for k in range(NP)])
    Wa4 = sum(jnp.pad(Wa, ((A * k, 128 - A * (k + 1)),
                           (k, NP - k - 1))) for k in range(NP))
    ctx, alpha, lp = pl.pallas_call(
        _body,
        grid=(B // G,),
        in_specs=[
            pl.BlockSpec((G, L, D), lambda b: (b, 0, 0)),
            pl.BlockSpec((G, H), lambda b: (b, 0)),
            pl.BlockSpec((NP, D, 128), lambda b: (0, 0, 0)),
            pl.BlockSpec((128, NP), lambda b: (0, 0)),
            pl.BlockSpec((1, A), lambda b: (0, 0)),
            pl.BlockSpec((H, A), lambda b: (0, 0)),
            pl.BlockSpec((1, A), lambda b: (0, 0)),
            pl.BlockSpec((1, 1), lambda b: (0, 0)),
            pl.BlockSpec((H, 1), lambda b: (0, 0)),
            pl.BlockSpec((1, 1), lambda b: (0, 0)),
        ],
        out_specs=[
            pl.BlockSpec((G, D), lambda b: (b, 0)),
            pl.BlockSpec((G, L), lambda b: (b, 0)),
            pl.BlockSpec((G, 1), lambda b: (b, 0)),
        ],
        out_shape=[
            jax.ShapeDtypeStruct((B, D), f32),
            jax.ShapeDtypeStruct((B, L), f32),
            jax.ShapeDtypeStruct((B, 1), f32),
        ],
    )(features, hidden, W4, Wa4, bf.reshape(1, A), Wh,
      bh.reshape(1, A), ba.reshape(1, 1), Wb, bb.reshape(1, 1))
    return ctx, alpha, lp.reshape(B)
